# Initial kernel scaffold; baseline (speedup 1.0000x reference)
import jax
import jax.numpy as jnp
from jax.experimental import pallas as pl


def kernel(x, Wp, bp, Wg, W1, b1, W2, b2, Wc, bc):
    raise NotImplementedError("write your pallas kernel here")


# speedup vs baseline: 3.1341x; 3.1341x over previous
import functools

import jax
import jax.numpy as jnp
from jax.experimental import pallas as pl
from jax.experimental.pallas import tpu as pltpu

TOKENS = 8192
IN_FEATURES = 1024
HIDDEN = 1024
N_CLASSES = 1000
N_EXPERTS = 8
TOP_K = 2
EXPERT_DIM = 256

TM = 512
EPAD = 128
CPAD = 1024


def _moe_body(x_ref, wpt_ref, bp_ref, wgt_ref, w1t_ref, b1_ref, w2t_ref,
              b2_ref, wct_ref, bc_ref, probs_ref, cls_ref):
    x = x_ref[...]
    h = jnp.dot(x, wpt_ref[...], preferred_element_type=jnp.float32)
    h = jnp.maximum(h + bp_ref[...], 0.0)

    logits = jnp.dot(h, wgt_ref[...], preferred_element_type=jnp.float32)
    col = jax.lax.broadcasted_iota(jnp.int32, (TM, EPAD), 1)
    neg = jnp.float32(-1e30)
    logits = jnp.where(col < N_EXPERTS, logits, neg)
    lmax = jnp.max(logits, axis=1, keepdims=True)
    ex = jnp.exp(logits - lmax)
    probs = ex / jnp.sum(ex, axis=1, keepdims=True)
    probs_ref[...] = probs

    w1 = jnp.max(probs, axis=1, keepdims=True)
    i1 = jnp.min(jnp.where(probs == w1, col, EPAD), axis=1, keepdims=True)
    probs2 = jnp.where(col == i1, -1.0, probs)
    w2 = jnp.max(probs2, axis=1, keepdims=True)
    i2 = jnp.min(jnp.where(probs2 == w2, col, EPAD), axis=1, keepdims=True)
    s = w1 + w2
    wd = jnp.where(col == i1, w1 / s, jnp.where(col == i2, w2 / s, 0.0))

    hb = h.astype(jnp.bfloat16)
    acc = jnp.zeros((TM, HIDDEN), dtype=jnp.float32)
    for e in range(N_EXPERTS):
        hid = jnp.dot(hb, w1t_ref[e], preferred_element_type=jnp.float32)
        hid = jnp.maximum(hid + b1_ref[e][None, :], 0.0)
        out_e = jnp.dot(hid.astype(jnp.bfloat16), w2t_ref[e],
                        preferred_element_type=jnp.float32)
        out_e = out_e + b2_ref[e][None, :]
        acc = acc + wd[:, e:e + 1] * out_e

    h2 = jnp.maximum(acc, 0.0).astype(jnp.bfloat16)
    cls = jnp.dot(h2, wct_ref[...], preferred_element_type=jnp.float32)
    cls_ref[...] = cls + bc_ref[...]


@jax.jit
def _run(x, WpT, bp, WgT, W1T, b1, W2T, b2, WcT, bc):
    grid = (TOKENS // TM,)
    full = lambda *shape: pl.BlockSpec(shape, lambda i: (0,) * len(shape))
    probs_pad, cls_pad = pl.pallas_call(
        _moe_body,
        grid=grid,
        in_specs=[
            pl.BlockSpec((TM, IN_FEATURES), lambda i: (i, 0)),
            full(IN_FEATURES, HIDDEN),
            full(1, HIDDEN),
            full(HIDDEN, EPAD),
            full(N_EXPERTS, HIDDEN, EXPERT_DIM),
            full(N_EXPERTS, EXPERT_DIM),
            full(N_EXPERTS, EXPERT_DIM, HIDDEN),
            full(N_EXPERTS, HIDDEN),
            full(HIDDEN, CPAD),
            full(1, CPAD),
        ],
        out_specs=[
            pl.BlockSpec((TM, EPAD), lambda i: (i, 0)),
            pl.BlockSpec((TM, CPAD), lambda i: (i, 0)),
        ],
        out_shape=[
            jax.ShapeDtypeStruct((TOKENS, EPAD), jnp.float32),
            jax.ShapeDtypeStruct((TOKENS, CPAD), jnp.float32),
        ],
    )(x, WpT, bp, WgT, W1T, b1, W2T, b2, WcT, bc)
    return probs_pad, cls_pad


def kernel(x, Wp, bp, Wg, W1, b1, W2, b2, Wc, bc):
    WpT = Wp.T
    WgT = jnp.zeros((HIDDEN, EPAD), jnp.float32).at[:, :N_EXPERTS].set(Wg.T)
    W1T = jnp.transpose(W1, (0, 2, 1)).astype(jnp.bfloat16)
    W2T = jnp.transpose(W2, (0, 2, 1)).astype(jnp.bfloat16)
    WcT = jnp.zeros((HIDDEN, CPAD), jnp.bfloat16).at[:, :N_CLASSES].set(
        Wc.T.astype(jnp.bfloat16))
    bcp = jnp.zeros((1, CPAD), jnp.float32).at[0, :N_CLASSES].set(bc)
    probs_pad, cls_pad = _run(x, WpT, bp[None, :], WgT, W1T, b1, W2T, b2,
                              WcT, bcp)
    return cls_pad[:, :N_CLASSES], probs_pad[:, :N_EXPERTS]


# speedup vs baseline: 3.5458x; 1.1314x over previous
import jax
import jax.numpy as jnp
from jax.experimental import pallas as pl

TOKENS = 8192
IN_FEATURES = 1024
HIDDEN = 1024
N_CLASSES = 1000
N_EXPERTS = 8
TOP_K = 2
EXPERT_DIM = 256

TM = 512
EPAD = 128

_NT = (((1,), (1,)), ((), ()))


def _moe_body(x_ref, wp_ref, bp_ref, wg_ref, w1_ref, b1_ref, w2_ref,
              b2_ref, wc_ref, bc_ref, probs_ref, cls_ref):
    x = x_ref[...]
    h = jax.lax.dot_general(x, wp_ref[...], _NT,
                            preferred_element_type=jnp.float32)
    h = jnp.maximum(h + bp_ref[...], 0.0)

    logits = jax.lax.dot_general(h, wg_ref[...], _NT,
                                 preferred_element_type=jnp.float32)
    col = jax.lax.broadcasted_iota(jnp.int32, (TM, EPAD), 1)
    logits = jnp.where(col < N_EXPERTS, logits, jnp.float32(-1e30))
    lmax = jnp.max(logits, axis=1, keepdims=True)
    ex = jnp.exp(logits - lmax)
    probs = ex / jnp.sum(ex, axis=1, keepdims=True)
    probs_ref[...] = probs

    w1 = jnp.max(probs, axis=1, keepdims=True)
    i1 = jnp.min(jnp.where(probs == w1, col, EPAD), axis=1, keepdims=True)
    probs2 = jnp.where(col == i1, -1.0, probs)
    w2 = jnp.max(probs2, axis=1, keepdims=True)
    i2 = jnp.min(jnp.where(probs2 == w2, col, EPAD), axis=1, keepdims=True)
    s = w1 + w2
    wd = jnp.where(col == i1, w1 / s, jnp.where(col == i2, w2 / s, 0.0))

    hb = h.astype(jnp.bfloat16)
    acc = jnp.zeros((TM, HIDDEN), dtype=jnp.float32)
    for e in range(N_EXPERTS):
        hid = jax.lax.dot_general(hb, w1_ref[e], _NT,
                                  preferred_element_type=jnp.float32)
        hid = jnp.maximum(hid + b1_ref[e][None, :], 0.0)
        out_e = jax.lax.dot_general(hid.astype(jnp.bfloat16), w2_ref[e], _NT,
                                    preferred_element_type=jnp.float32)
        out_e = out_e + b2_ref[e][None, :]
        acc = acc + wd[:, e:e + 1] * out_e

    h2 = jnp.maximum(acc, 0.0).astype(jnp.bfloat16)
    cls = jax.lax.dot_general(h2, wc_ref[...], _NT,
                              preferred_element_type=jnp.float32)
    cls_ref[...] = cls + bc_ref[...]


@jax.jit
def _run(x, Wp, bp, Wg_pad, W1b, b1, W2b, b2, Wcb, bc):
    grid = (TOKENS // TM,)
    full = lambda *shape: pl.BlockSpec(shape, lambda i: (0,) * len(shape))
    probs_pad, cls = pl.pallas_call(
        _moe_body,
        grid=grid,
        in_specs=[
            pl.BlockSpec((TM, IN_FEATURES), lambda i: (i, 0)),
            full(HIDDEN, IN_FEATURES),
            full(1, HIDDEN),
            full(EPAD, HIDDEN),
            full(N_EXPERTS, EXPERT_DIM, HIDDEN),
            full(N_EXPERTS, EXPERT_DIM),
            full(N_EXPERTS, HIDDEN, EXPERT_DIM),
            full(N_EXPERTS, HIDDEN),
            full(N_CLASSES, HIDDEN),
            full(1, N_CLASSES),
        ],
        out_specs=[
            pl.BlockSpec((TM, EPAD), lambda i: (i, 0)),
            pl.BlockSpec((TM, N_CLASSES), lambda i: (i, 0)),
        ],
        out_shape=[
            jax.ShapeDtypeStruct((TOKENS, EPAD), jnp.float32),
            jax.ShapeDtypeStruct((TOKENS, N_CLASSES), jnp.float32),
        ],
    )(x, Wp, bp, Wg_pad, W1b, b1, W2b, b2, Wcb, bc)
    return probs_pad, cls


def kernel(x, Wp, bp, Wg, W1, b1, W2, b2, Wc, bc):
    Wg_pad = jnp.zeros((EPAD, HIDDEN), jnp.float32).at[:N_EXPERTS].set(Wg)
    probs_pad, cls = _run(x, Wp, bp[None, :], Wg_pad,
                          W1.astype(jnp.bfloat16), b1,
                          W2.astype(jnp.bfloat16), b2,
                          Wc.astype(jnp.bfloat16), bc[None, :])
    return cls, probs_pad[:, :N_EXPERTS]
